# baseline (speedup 1.0000x reference)
import functools

import jax
import jax.numpy as jnp
from jax import lax
from jax.experimental import pallas as pl
from jax.experimental.pallas import tpu as pltpu
from jax.experimental.pallas import tpu_sc as plsc

_VOCAB = 1000000
_DIM = 64
_B = 16384
_NEG = 20

_NC = 2
_NS = 16
_NW = _NC * _NS
_BW = _B // _NW

_SBV = 128
_NSB = 999936 // _SBV
_SB_ITERS = -(-_NSB // _NW)
_PAIRS = _VOCAB // 2
_TAIL_P0 = 999936 // 2

_C = 16
_NCHUNK = _BW // _C
_ROWS = _C * _NEG


def _transpose_sb(st_in, st_out, iota):
    iota_half = iota >> 1
    par64v = (iota & 1) * 64
    for c in range(_SBV // 16):
        rowvec = iota_half + c * 8

        def d_body(d, carry, rowvec=rowvec, c=c):
            x = st_in[d, pl.ds(c * 16, 16)]
            plsc.store_scatter(st_out, [rowvec, par64v + d], x)
            return carry

        lax.fori_loop(0, 64, d_body, 0, unroll=8)


def _detile_body(embT, oembT, tail_e, tail_o, s_e, s_o,
                 st_in_e, st_in_o, st_out, sem_e, sem_o):
    c = lax.axis_index("c")
    s = lax.axis_index("s")
    wid = s * _NC + c
    iota = jnp.arange(16, dtype=jnp.int32)

    def sb_body(i, carry):
        sb = wid + i * _NW

        @pl.when(sb < _NSB)
        def _():
            v0 = sb * _SBV
            p0 = sb * (_SBV // 2)
            cp_e = pltpu.async_copy(
                embT.at[pl.ds(0, _DIM), pl.ds(v0, _SBV)], st_in_e, sem_e)
            cp_o = pltpu.async_copy(
                oembT.at[pl.ds(0, _DIM), pl.ds(v0, _SBV)], st_in_o, sem_o)
            cp_e.wait()
            _transpose_sb(st_in_e, st_out, iota)
            pltpu.sync_copy(st_out, s_e.at[pl.ds(p0, _SBV // 2), pl.ds(0, 128)])
            cp_o.wait()
            _transpose_sb(st_in_o, st_out, iota)
            pltpu.sync_copy(st_out, s_o.at[pl.ds(p0, _SBV // 2), pl.ds(0, 128)])

        return carry

    lax.fori_loop(0, _SB_ITERS, sb_body, 0, unroll=False)

    @pl.when(wid == 0)
    def _():
        pltpu.sync_copy(tail_e, st_out.at[pl.ds(0, 32), pl.ds(0, 128)])
        pltpu.sync_copy(st_out.at[pl.ds(0, 32), pl.ds(0, 128)],
                        s_e.at[pl.ds(_TAIL_P0, 32), pl.ds(0, 128)])
        pltpu.sync_copy(tail_o, st_out.at[pl.ds(0, 32), pl.ds(0, 128)])
        pltpu.sync_copy(st_out.at[pl.ds(0, 32), pl.ds(0, 128)],
                        s_o.at[pl.ds(_TAIL_P0, 32), pl.ds(0, 128)])


def _scores_body(cen_pr, ctx_pr, neg_pr, cen_pa, ctx_pa, neg_pa, s_all,
                 scores_out,
                 pr_cen_v, pr_ctx_v, pr_neg_v, pa_cen_v, pa_ctx_v, pa_neg_v,
                 g_cen, g_ctx, g_neg, r_cen, r_ctx, r_neg,
                 part_v, scores_s, sem):
    c = lax.axis_index("c")
    s = lax.axis_index("s")
    wid = s * _NC + c
    base = wid * _BW
    iota = jnp.arange(16, dtype=jnp.int32)

    pltpu.sync_copy(cen_pr.at[pl.ds(base, _BW)], pr_cen_v)
    pltpu.sync_copy(ctx_pr.at[pl.ds(base, _BW)], pr_ctx_v)
    pltpu.sync_copy(neg_pr.at[pl.ds(base * _NEG, _BW * _NEG)], pr_neg_v)
    pltpu.sync_copy(cen_pa.at[pl.ds(base, _BW)], pa_cen_v.at[pl.ds(0, _BW)])
    pltpu.sync_copy(ctx_pa.at[pl.ds(base, _BW)], pa_ctx_v.at[pl.ds(0, _BW)])
    pltpu.sync_copy(neg_pa.at[pl.ds(base * _NEG, _BW * _NEG)], pa_neg_v)
    pa_cen_v[pl.ds(_BW, 16)] = jnp.zeros((16,), jnp.int32)
    pa_ctx_v[pl.ds(_BW, 16)] = jnp.zeros((16,), jnp.int32)

    def chunk_body(ci, carry):
        cb = ci * _C
        nb = ci * _ROWS
        cps = [
            pltpu.async_copy(s_all.at[pr_cen_v.at[pl.ds(cb, _C)]], g_cen, sem),
            pltpu.async_copy(s_all.at[pr_ctx_v.at[pl.ds(cb, _C)]], g_ctx, sem),
        ] + [
            pltpu.async_copy(s_all.at[pr_neg_v.at[pl.ds(nb + off, sz)]],
                             g_neg.at[pl.ds(off, sz)], sem)
            for off, sz in ((0, 128), (128, 128), (256, 64))
        ]
        for cp in cps:
            cp.wait()

        pv_cen = [pa_cen_v[pl.ds(cb + h * 16, 16)] for h in range(_C // 16)]
        pv_ctx = [pa_ctx_v[pl.ds(cb + h * 16, 16)] for h in range(_C // 16)]
        for i in range(_C):
            pc = pv_cen[i // 16][i % 16]
            px = pv_ctx[i // 16][i % 16]
            for r in range(4):
                r_cen[pl.ds(i * 64 + r * 16, 16)] = g_cen[i, pl.ds(pc + r * 16, 16)]
                r_ctx[pl.ds(i * 64 + r * 16, 16)] = g_ctx[i, pl.ds(px + r * 16, 16)]

        def grp_body(grp, carry2):
            pv = pa_neg_v[pl.ds(nb + grp * 16, 16)]
            row0 = grp * 16
            for j in range(16):
                pn = pv[j]
                row = row0 + j
                d0 = row * 64
                for r in range(4):
                    r_neg[pl.ds(d0 + r * 16, 16)] = g_neg[row, pl.ds(pn + r * 16, 16)]
            return carry2

        lax.fori_loop(0, _ROWS // 16, grp_body, 0, unroll=False)

        def b_body(b, carry2):
            cb64 = b * 64
            c0 = r_cen[pl.ds(cb64, 16)]
            c1 = r_cen[pl.ds(cb64 + 16, 16)]
            c2 = r_cen[pl.ds(cb64 + 32, 16)]
            c3 = r_cen[pl.ds(cb64 + 48, 16)]
            for k in range(_NEG):
                r = (b * _NEG + k) * 64
                t = (c0 * r_neg[pl.ds(r, 16)]
                     + c1 * r_neg[pl.ds(r + 16, 16)]
                     + c2 * r_neg[pl.ds(r + 32, 16)]
                     + c3 * r_neg[pl.ds(r + 48, 16)])
                part_v[pl.ds(k * 16, 16)] = plsc.cumsum(t)
            p = (c0 * r_ctx[pl.ds(cb64, 16)]
                 + c1 * r_ctx[pl.ds(cb64 + 16, 16)]
                 + c2 * r_ctx[pl.ds(cb64 + 32, 16)]
                 + c3 * r_ctx[pl.ds(cb64 + 48, 16)])
            part_v[pl.ds(_NEG * 16, 16)] = plsc.cumsum(p)
            g1 = plsc.load_gather(part_v, [iota * 16 + 15])
            g2 = plsc.load_gather(part_v, [iota * 16 + 271])
            bb32 = (cb + b) * 32
            scores_s[pl.ds(bb32, 16)] = g1
            scores_s[pl.ds(bb32 + 16, 16)] = g2
            return carry2

        lax.fori_loop(0, _C, b_body, 0, unroll=False)
        return carry

    lax.fori_loop(0, _NCHUNK, chunk_body, 0, unroll=False)

    pltpu.sync_copy(scores_s, scores_out.at[pl.ds(base * 32, _BW * 32)])


@jax.jit
def _sc_call(cen, ctx, neg, embedding, output_embedding):
    mesh = plsc.VectorSubcoreMesh(core_axis_name="c", subcore_axis_name="s")
    params = pltpu.CompilerParams(
        needs_layout_passes=False, use_tc_tiling_on_sc=True)

    s_all = jnp.concatenate(
        [embedding.reshape(_PAIRS, 128),
         output_embedding.reshape(_PAIRS, 128)], axis=0)

    scores = pl.kernel(
        _scores_body,
        out_type=jax.ShapeDtypeStruct((_B * 32,), jnp.float32),
        mesh=mesh,
        scratch_types=[
            pltpu.VMEM((_BW,), jnp.int32),
            pltpu.VMEM((_BW,), jnp.int32),
            pltpu.VMEM((_BW * _NEG,), jnp.int32),
            pltpu.VMEM((_BW + 16,), jnp.int32),
            pltpu.VMEM((_BW + 16,), jnp.int32),
            pltpu.VMEM((_BW * _NEG,), jnp.int32),
            pltpu.VMEM((_C, 128), jnp.float32),
            pltpu.VMEM((_C, 128), jnp.float32),
            pltpu.VMEM((_ROWS, 128), jnp.float32),
            pltpu.VMEM((_C * _DIM,), jnp.float32),
            pltpu.VMEM((_C * _DIM,), jnp.float32),
            pltpu.VMEM((_ROWS * _DIM,), jnp.float32),
            pltpu.VMEM((512,), jnp.float32),
            pltpu.VMEM((_BW * 32,), jnp.float32),
            pltpu.SemaphoreType.DMA,
        ],
        compiler_params=params,
    )(cen >> 1, (ctx >> 1) + _PAIRS, (neg >> 1) + _PAIRS,
      (cen & 1) * 64, (ctx & 1) * 64, (neg & 1) * 64,
      s_all)
    return scores


def kernel(center, context, negatives, embedding, output_embedding):
    cen = center.astype(jnp.int32)
    ctx = context.astype(jnp.int32)
    neg = negatives.astype(jnp.int32).reshape(-1)
    scores = _sc_call(cen, ctx, neg, embedding, output_embedding)
    scores = scores.reshape(_B, 32)
    return scores[:, 20], scores[:, :20]


# speedup vs baseline: 1.4349x; 1.4349x over previous
import functools

import jax
import jax.numpy as jnp
from jax import lax
from jax.experimental import pallas as pl
from jax.experimental.pallas import tpu as pltpu
from jax.experimental.pallas import tpu_sc as plsc

_VOCAB = 1000000
_DIM = 64
_B = 16384
_NEG = 20

_NC = 2
_NS = 16
_NW = _NC * _NS
_BW = _B // _NW
_C = 32
_NCHUNK = _BW // _C
_NEG_GATHER = 128
_NEG_STEPS = (_C * _NEG) // _NEG_GATHER


def _sc_body(cen_idx, ctx_idx, neg_idx, emb, oemb, scores_out,
             idx_cen_v, idx_ctx_v, idx_neg_v,
             cen_a, ctx_a, neg_a, cen_b, ctx_b, neg_b,
             part_v, scores_s, sem_a, sem_b):
    c = lax.axis_index("c")
    s = lax.axis_index("s")
    wid = s * _NC + c
    base = wid * _BW

    pltpu.sync_copy(cen_idx.at[pl.ds(base, _BW)], idx_cen_v)
    pltpu.sync_copy(ctx_idx.at[pl.ds(base, _BW)], idx_ctx_v)
    pltpu.sync_copy(neg_idx.at[pl.ds(base * _NEG, _BW * _NEG)], idx_neg_v)

    iota = jnp.arange(16, dtype=jnp.int32)

    def fire(ci, cen_v, ctx_v, neg_v, sem):
        cb = ci * _C
        pltpu.async_copy(emb.at[idx_cen_v.at[pl.ds(cb, _C)]], cen_v, sem)
        pltpu.async_copy(oemb.at[idx_ctx_v.at[pl.ds(cb, _C)]], ctx_v, sem)
        for j in range(_NEG_STEPS):
            pltpu.async_copy(
                oemb.at[idx_neg_v.at[pl.ds(cb * _NEG + j * _NEG_GATHER,
                                           _NEG_GATHER)]],
                neg_v.at[pl.ds(j * _NEG_GATHER, _NEG_GATHER)], sem)

    def drain(cen_v, ctx_v, neg_v, sem):
        pltpu.make_async_copy(emb.at[pl.ds(0, _C)], cen_v, sem).wait()
        pltpu.make_async_copy(oemb.at[pl.ds(0, _C)], ctx_v, sem).wait()
        for j in range(_NEG_STEPS):
            pltpu.make_async_copy(
                oemb.at[pl.ds(0, _NEG_GATHER)],
                neg_v.at[pl.ds(j * _NEG_GATHER, _NEG_GATHER)], sem).wait()

    def compute(ci, cen_v, ctx_v, neg_v):
        cb = ci * _C

        def b_body(b, carry2):
            pb = (b & 1) << 9
            c0 = cen_v[b, pl.ds(0, 16)]
            c1 = cen_v[b, pl.ds(16, 16)]
            c2 = cen_v[b, pl.ds(32, 16)]
            c3 = cen_v[b, pl.ds(48, 16)]
            for k in range(_NEG):
                r = b * _NEG + k
                t = (c0 * neg_v[r, pl.ds(0, 16)]
                     + c1 * neg_v[r, pl.ds(16, 16)]
                     + c2 * neg_v[r, pl.ds(32, 16)]
                     + c3 * neg_v[r, pl.ds(48, 16)])
                part_v[pl.ds(pb + k * 16, 16)] = plsc.cumsum(t)
            p = (c0 * ctx_v[b, pl.ds(0, 16)]
                 + c1 * ctx_v[b, pl.ds(16, 16)]
                 + c2 * ctx_v[b, pl.ds(32, 16)]
                 + c3 * ctx_v[b, pl.ds(48, 16)])
            part_v[pl.ds(pb + _NEG * 16, 16)] = plsc.cumsum(p)
            g1 = plsc.load_gather(part_v, [pb + iota * 16 + 15])
            g2 = plsc.load_gather(part_v, [pb + iota * 16 + 271])
            bb = cb + b
            scores_s[bb, pl.ds(0, 16)] = g1
            scores_s[bb, pl.ds(16, 16)] = g2
            return carry2

        lax.fori_loop(0, _C, b_body, 0, unroll=2)

    fire(0, cen_a, ctx_a, neg_a, sem_a)

    def super_body(h, carry):
        ci = h * 2
        fire(ci + 1, cen_b, ctx_b, neg_b, sem_b)
        drain(cen_a, ctx_a, neg_a, sem_a)
        compute(ci, cen_a, ctx_a, neg_a)

        @pl.when(ci + 2 < _NCHUNK)
        def _():
            fire(ci + 2, cen_a, ctx_a, neg_a, sem_a)

        drain(cen_b, ctx_b, neg_b, sem_b)
        compute(ci + 1, cen_b, ctx_b, neg_b)
        return carry

    lax.fori_loop(0, _NCHUNK // 2, super_body, 0, unroll=False)

    pltpu.sync_copy(scores_s, scores_out.at[pl.ds(base, _BW)])


@jax.jit
def _sc_call(cen_idx, ctx_idx, neg_idx, emb, oemb):
    mesh = plsc.VectorSubcoreMesh(core_axis_name="c", subcore_axis_name="s")
    return pl.kernel(
        _sc_body,
        out_type=jax.ShapeDtypeStruct((_B, 32), jnp.float32),
        mesh=mesh,
        scratch_types=[
            pltpu.VMEM((_BW,), jnp.int32),
            pltpu.VMEM((_BW,), jnp.int32),
            pltpu.VMEM((_BW * _NEG,), jnp.int32),
            pltpu.VMEM((_C, _DIM), jnp.float32),
            pltpu.VMEM((_C, _DIM), jnp.float32),
            pltpu.VMEM((_C * _NEG, _DIM), jnp.float32),
            pltpu.VMEM((_C, _DIM), jnp.float32),
            pltpu.VMEM((_C, _DIM), jnp.float32),
            pltpu.VMEM((_C * _NEG, _DIM), jnp.float32),
            pltpu.VMEM((1024,), jnp.float32),
            pltpu.VMEM((_BW, 32), jnp.float32),
            pltpu.SemaphoreType.DMA,
            pltpu.SemaphoreType.DMA,
        ],
        compiler_params=pltpu.CompilerParams(
            needs_layout_passes=False, use_tc_tiling_on_sc=False),
    )(cen_idx, ctx_idx, neg_idx, emb, oemb)


def kernel(center, context, negatives, embedding, output_embedding):
    cen = center.astype(jnp.int32)
    ctx = context.astype(jnp.int32)
    neg = negatives.astype(jnp.int32).reshape(-1)
    scores = _sc_call(cen, ctx, neg, embedding, output_embedding)
    return scores[:, 20], scores[:, :20]
